# baseline (speedup 1.0000x reference)
import jax
import jax.numpy as jnp
from jax.experimental import pallas as pl
from jax.experimental.pallas import tpu as pltpu

_B, _T, _N, _L = 64, 1000, 256, 128
_NEG = -1e30
_BH = 64
_TC = 40
_NT = _T // _TC
_BF = jnp.bfloat16
_F32 = jnp.float32


def _emit_dot(xslab_bf, oh_b, msr, addr):
    pltpu.matmul_push_rhs(oh_b, staging_register=msr, mxu_index=1)
    pltpu.matmul_acc_lhs(addr, xslab_bf, 1, load_staged_rhs=msr)
    return pltpu.matmul_pop(addr, (48, _N), _F32, 1)


def _asg_kernel(xc_ref, xb_ref, xb0_ref, tgt_ref, ts_ref, trans_ref, out_ref,
                wfcc, zfcc, afac, tself, tmove, emits, oh_scr, gmax_s):
    t_idx = pl.program_id(0)
    zeros8 = jnp.zeros((8, _N), _BF)

    @pl.when(t_idx == 0)
    def _init():
        iota_nl = jax.lax.broadcasted_iota(jnp.int32, (_N, _L), 0)
        tr = trans_ref[...]
        gmax = jnp.max(tr)
        gmax_s[0, 0] = gmax
        pltpu.matmul_push_rhs(jnp.exp(tr - gmax).astype(_BF),
                              staging_register=0, mxu_index=0, transpose=True)
        pltpu.matmul_acc_lhs(240, jnp.zeros((16, _N), _BF), 0,
                             load_staged_rhs=0)
        _ = pltpu.matmul_pop(240, (16, _N), _F32, 0)
        tr_bf = tr.astype(_BF)
        iota_l = jax.lax.broadcasted_iota(jnp.int32, (1, _L), 1)
        zcol = jnp.zeros((_N, _L), _BF)
        for b in range(_BH):
            ohf = (iota_nl == tgt_ref[b:b + 1, :]).astype(jnp.float32)
            oh_scr[b] = jnp.concatenate([ohf.astype(_BF), zcol], axis=1)
            pltpu.matmul_push_rhs(oh_scr[b], staging_register=b % 2,
                                  mxu_index=1)
            pltpu.matmul_acc_lhs(64 * (b % 2), tr_bf, 1,
                                 load_staged_rhs=b % 2)
            a = pltpu.matmul_pop(64 * (b % 2), (_N, _N), _F32, 1)[:, :_L]
            tself[b:b + 1, :] = jnp.sum(ohf * a, axis=0, keepdims=True)
            a_shift = jnp.concatenate(
                [jnp.zeros((_N, 1), jnp.float32), a[:, :-1]], axis=1)
            tm = jnp.sum(ohf * a_shift, axis=0, keepdims=True)
            tmove[b:b + 1, :] = jnp.where(iota_l == 0, _NEG, tm)
        for b in range(_BH):
            xslab = jnp.concatenate(
                [xb0_ref[b].astype(_BF), zeros8], axis=0)
            e = _emit_dot(xslab, oh_scr[b], b % 2, 32 * (b % 2))
            emits[0, :, b, :] = e[:_TC, :_L]
        x0 = xb0_ref[:, 0, :]
        z0 = jnp.max(x0, axis=1, keepdims=True)
        wfcc[...] = jnp.exp(x0 - z0)
        zfcc[...] = z0
        iota_bl = jax.lax.broadcasted_iota(jnp.int32, (_BH, _L), 1)
        afac[...] = jnp.where(iota_bl == 0, emits[0, 0], _NEG)

    sel = jax.lax.rem(t_idx, 2)
    sel_next = jax.lax.rem(t_idx + 1, 2)
    ts_mat = tself[...]
    tm_mat = tmove[...]
    neg_col = jnp.full((_BH, 1), _NEG, jnp.float32)

    w = wfcc[...].astype(_BF)
    z = zfcc[...]
    a_fac = afac[...]

    n_groups = _TC // 8
    splits = [(_BH * g) // n_groups for g in range(n_groups + 1)]

    for g in range(n_groups):
        xg = xc_ref[:, g * 8:(g + 1) * 8, :]
        for j in range(8):
            pltpu.matmul_acc_lhs(0, w, 0, load_staged_rhs=None)
            q = pltpu.matmul_pop(0, (_BH, _N), _F32, 0)
            scale = jnp.exp(xg[:, j, :])
            if j % 4 == 3:
                scale = scale * (1.0 / rs)
                z = z + jnp.log(rs)
            new_w = (scale * q).astype(_BF)
            stay = a_fac + ts_mat
            move = jnp.concatenate([neg_col, a_fac[:, :-1]], axis=1) + tm_mat
            mx = jnp.maximum(stay, move)
            new_fac = (mx + jnp.log1p(jnp.exp(-jnp.abs(stay - move)))
                       + emits[sel, g * 8 + j])
            if g == 0 and j == 0:
                new_w = jnp.where(t_idx == 0, w, new_w)
                new_fac = jnp.where(t_idx == 0, a_fac, new_fac)
            w, a_fac = new_w, new_fac
            if j % 4 == 2:
                rs = jnp.sum(w.astype(jnp.float32), axis=1, keepdims=True)
        for b in range(splits[g], splits[g + 1]):
            xslab = jnp.concatenate([xb_ref[b].astype(_BF), zeros8], axis=0)
            e = _emit_dot(xslab, oh_scr[b], b % 2, 32 * (b % 2))
            emits[sel_next, :, b, :] = e[:_TC, :_L]

    wfcc[...] = w.astype(jnp.float32)
    zfcc[...] = z
    afac[...] = a_fac

    @pl.when(t_idx == _NT - 1)
    def _finish():
        gmax = gmax_s[0, 0]
        fcc = (z + (_T - 1) * gmax
               + jnp.log(jnp.sum(w.astype(jnp.float32), axis=1,
                                 keepdims=True)))
        iota_bl = jax.lax.broadcasted_iota(jnp.int32, (_BH, _L), 1)
        mask = iota_bl == (ts_ref[...] - 1)
        fac = jnp.sum(jnp.where(mask, a_fac, 0.0), axis=1, keepdims=True)
        out_ref[...] = fcc - fac


def _asg_loss(x, target, target_size, trans):
    ts2 = target_size.reshape(_B, 1)
    out = pl.pallas_call(
        _asg_kernel,
        grid=(_NT,),
        in_specs=[
            pl.BlockSpec((_BH, _TC, _N), lambda t: (0, t, 0)),
            pl.BlockSpec((_BH, _TC, _N),
                         lambda t: (0, jnp.minimum(t + 1, _NT - 1), 0)),
            pl.BlockSpec((_BH, _TC, _N), lambda t: (0, 0, 0)),
            pl.BlockSpec((_BH, _L), lambda t: (0, 0)),
            pl.BlockSpec((_BH, 1), lambda t: (0, 0)),
            pl.BlockSpec((_N, _N), lambda t: (0, 0)),
        ],
        out_specs=pl.BlockSpec((_BH, 1), lambda t: (0, 0)),
        out_shape=jax.ShapeDtypeStruct((_B, 1), jnp.float32),
        scratch_shapes=[
            pltpu.VMEM((_BH, _N), jnp.float32),
            pltpu.VMEM((_BH, 1), jnp.float32),
            pltpu.VMEM((_BH, _L), jnp.float32),
            pltpu.VMEM((_BH, _L), jnp.float32),
            pltpu.VMEM((_BH, _L), jnp.float32),
            pltpu.VMEM((2, _TC, _BH, _L), jnp.float32),
            pltpu.VMEM((_BH, _N, _N), _BF),
            pltpu.SMEM((1, 1), jnp.float32),
        ],
        compiler_params=pltpu.CompilerParams(
            dimension_semantics=("arbitrary",),
            vmem_limit_bytes=56 * 1024 * 1024,
        ),
    )(x, x, x, target, ts2, trans)
    return out[:, 0]


def kernel(input, target, target_size, trans):
    return jax.jit(_asg_loss)(input, target, target_size, trans)


# speedup vs baseline: 1.1179x; 1.1179x over previous
import jax
import jax.numpy as jnp
from jax.experimental import pallas as pl
from jax.experimental.pallas import tpu as pltpu

_B, _T, _N, _L = 64, 1000, 256, 128
_NEG = -1e30
_BH = 64
_TC = 40
_NT = _T // _TC
_BF = jnp.bfloat16
_F32 = jnp.float32


def _emit_dot(xslab_bf, oh_b, msr, addr):
    pltpu.matmul_push_rhs(oh_b, staging_register=msr, mxu_index=1)
    pltpu.matmul_acc_lhs(addr, xslab_bf, 1, load_staged_rhs=msr)
    return pltpu.matmul_pop(addr, (48, _N), _F32, 1)


def _asg_kernel(xt_ref, xb_ref, xb0_ref, tgt_ref, ts_ref, trans_ref, out_ref,
                wfcc, zfcc, afac, tself, tmove, emits, oh_scr, gmax_s):
    t_idx = pl.program_id(0)
    zeros8 = jnp.zeros((8, _N), _BF)

    @pl.when(t_idx == 0)
    def _init():
        iota_nl = jax.lax.broadcasted_iota(jnp.int32, (_N, _L), 0)
        tr = trans_ref[...]
        gmax = jnp.max(tr)
        gmax_s[0, 0] = gmax
        pltpu.matmul_push_rhs(jnp.exp(tr - gmax).astype(_BF),
                              staging_register=0, mxu_index=0, transpose=True)
        pltpu.matmul_acc_lhs(240, jnp.zeros((16, _N), _BF), 0,
                             load_staged_rhs=0)
        _ = pltpu.matmul_pop(240, (16, _N), _F32, 0)
        tr_bf = tr.astype(_BF)
        iota_l = jax.lax.broadcasted_iota(jnp.int32, (1, _L), 1)
        zcol = jnp.zeros((_N, _L), _BF)
        for b in range(_BH):
            ohf = (iota_nl == tgt_ref[b:b + 1, :]).astype(jnp.float32)
            oh_scr[b] = jnp.concatenate([ohf.astype(_BF), zcol], axis=1)
            pltpu.matmul_push_rhs(oh_scr[b], staging_register=b % 2,
                                  mxu_index=1)
            pltpu.matmul_acc_lhs(64 * (b % 2), tr_bf, 1,
                                 load_staged_rhs=b % 2)
            a = pltpu.matmul_pop(64 * (b % 2), (_N, _N), _F32, 1)[:, :_L]
            tself[b:b + 1, :] = jnp.sum(ohf * a, axis=0, keepdims=True)
            a_shift = jnp.concatenate(
                [jnp.zeros((_N, 1), jnp.float32), a[:, :-1]], axis=1)
            tm = jnp.sum(ohf * a_shift, axis=0, keepdims=True)
            tmove[b:b + 1, :] = jnp.where(iota_l == 0, _NEG, tm)
        for b in range(_BH):
            xslab = jnp.concatenate(
                [xb0_ref[b].astype(_BF), zeros8], axis=0)
            e = _emit_dot(xslab, oh_scr[b], b % 2, 32 * (b % 2))
            emits[0, :, b, :] = e[:_TC, :_L]
        x0 = xt_ref[0].astype(jnp.float32)
        z0 = jnp.max(x0, axis=1, keepdims=True)
        wfcc[...] = jnp.exp(x0 - z0)
        zfcc[...] = z0
        iota_bl = jax.lax.broadcasted_iota(jnp.int32, (_BH, _L), 1)
        afac[...] = jnp.where(iota_bl == 0, emits[0, 0], _NEG)

    sel = jax.lax.rem(t_idx, 2)
    sel_next = jax.lax.rem(t_idx + 1, 2)
    ts_mat = tself[...]
    tm_mat = tmove[...]
    neg_col = jnp.full((_BH, 1), _NEG, jnp.float32)

    w = wfcc[...].astype(_BF)
    z = zfcc[...]
    a_fac = afac[...]

    n_groups = _TC // 8
    splits = [(_BH * g) // n_groups for g in range(n_groups + 1)]

    for g in range(n_groups):
        xg = xt_ref[g * 8:(g + 1) * 8]
        for j in range(8):
            pltpu.matmul_acc_lhs(0, w, 0, load_staged_rhs=None)
            q = pltpu.matmul_pop(0, (_BH, _N), _F32, 0)
            scale = jnp.exp(xg[j].astype(jnp.float32))
            if j % 4 == 3:
                scale = scale * (1.0 / rs)
                z = z + jnp.log(rs)
            new_w = (scale * q).astype(_BF)
            stay = a_fac + ts_mat
            move = jnp.concatenate([neg_col, a_fac[:, :-1]], axis=1) + tm_mat
            mx = jnp.maximum(stay, move)
            new_fac = (mx + jnp.log1p(jnp.exp(-jnp.abs(stay - move)))
                       + emits[sel, g * 8 + j])
            if g == 0 and j == 0:
                new_w = jnp.where(t_idx == 0, w, new_w)
                new_fac = jnp.where(t_idx == 0, a_fac, new_fac)
            w, a_fac = new_w, new_fac
            if j % 4 == 2:
                rs = jnp.sum(w.astype(jnp.float32), axis=1, keepdims=True)
        for b in range(splits[g], splits[g + 1]):
            xslab = jnp.concatenate([xb_ref[b].astype(_BF), zeros8], axis=0)
            e = _emit_dot(xslab, oh_scr[b], b % 2, 32 * (b % 2))
            emits[sel_next, :, b, :] = e[:_TC, :_L]

    wfcc[...] = w.astype(jnp.float32)
    zfcc[...] = z
    afac[...] = a_fac

    @pl.when(t_idx == _NT - 1)
    def _finish():
        gmax = gmax_s[0, 0]
        fcc = (z + (_T - 1) * gmax
               + jnp.log(jnp.sum(w.astype(jnp.float32), axis=1,
                                 keepdims=True)))
        iota_bl = jax.lax.broadcasted_iota(jnp.int32, (_BH, _L), 1)
        mask = iota_bl == (ts_ref[...] - 1)
        fac = jnp.sum(jnp.where(mask, a_fac, 0.0), axis=1, keepdims=True)
        out_ref[...] = fcc - fac


def _asg_loss(x, target, target_size, trans):
    xt = jnp.moveaxis(x, 1, 0).astype(jnp.bfloat16)
    ts2 = target_size.reshape(_B, 1)
    out = pl.pallas_call(
        _asg_kernel,
        grid=(_NT,),
        in_specs=[
            pl.BlockSpec((_TC, _BH, _N), lambda t: (t, 0, 0)),
            pl.BlockSpec((_BH, _TC, _N),
                         lambda t: (0, jnp.minimum(t + 1, _NT - 1), 0)),
            pl.BlockSpec((_BH, _TC, _N), lambda t: (0, 0, 0)),
            pl.BlockSpec((_BH, _L), lambda t: (0, 0)),
            pl.BlockSpec((_BH, 1), lambda t: (0, 0)),
            pl.BlockSpec((_N, _N), lambda t: (0, 0)),
        ],
        out_specs=pl.BlockSpec((_BH, 1), lambda t: (0, 0)),
        out_shape=jax.ShapeDtypeStruct((_B, 1), jnp.float32),
        scratch_shapes=[
            pltpu.VMEM((_BH, _N), jnp.float32),
            pltpu.VMEM((_BH, 1), jnp.float32),
            pltpu.VMEM((_BH, _L), jnp.float32),
            pltpu.VMEM((_BH, _L), jnp.float32),
            pltpu.VMEM((_BH, _L), jnp.float32),
            pltpu.VMEM((2, _TC, _BH, _L), jnp.float32),
            pltpu.VMEM((_BH, _N, _N), _BF),
            pltpu.SMEM((1, 1), jnp.float32),
        ],
        compiler_params=pltpu.CompilerParams(
            dimension_semantics=("arbitrary",),
            vmem_limit_bytes=56 * 1024 * 1024,
        ),
    )(xt, x, x, target, ts2, trans)
    return out[:, 0]


def kernel(input, target, target_size, trans):
    return jax.jit(_asg_loss)(input, target, target_size, trans)


# speedup vs baseline: 1.1251x; 1.0064x over previous
import jax
import jax.numpy as jnp
from jax.experimental import pallas as pl
from jax.experimental.pallas import tpu as pltpu

_B, _T, _N, _L = 64, 1000, 256, 128
_NEG = -1e30
_BH = 64
_TC = 40
_NT = _T // _TC
_BF = jnp.bfloat16
_F32 = jnp.float32


def _emit_dot(xslab_bf, oh_b, msr, addr):
    pltpu.matmul_push_rhs(oh_b, staging_register=msr, mxu_index=1)
    pltpu.matmul_acc_lhs(addr, xslab_bf, 1, load_staged_rhs=msr)
    return pltpu.matmul_pop(addr, (48, _N), _F32, 1)


def _asg_kernel(xt_ref, xb_ref, xb0_ref, tgt_ref, ts_ref, trans_ref, out_ref,
                wfcc, zfcc, afac, tself, tmove, emits, oh_scr, gmax_s):
    t_idx = pl.program_id(0)
    zeros8 = jnp.zeros((8, _N), _BF)

    @pl.when(t_idx == 0)
    def _init():
        iota_nl = jax.lax.broadcasted_iota(jnp.int32, (_N, _L), 0)
        tr = trans_ref[...]
        gmax = jnp.max(tr)
        gmax_s[0, 0] = gmax
        pltpu.matmul_push_rhs(jnp.exp(tr - gmax).astype(_BF),
                              staging_register=0, mxu_index=0, transpose=True)
        pltpu.matmul_acc_lhs(240, jnp.zeros((16, _N), _BF), 0,
                             load_staged_rhs=0)
        _ = pltpu.matmul_pop(240, (16, _N), _F32, 0)
        tr_bf = tr.astype(_BF)
        iota_l = jax.lax.broadcasted_iota(jnp.int32, (1, _L), 1)
        zcol = jnp.zeros((_N, _L), _BF)
        zcol_f = jnp.zeros((_N, 1), jnp.float32)
        for b in range(0, _BH, 2):
            ohf0 = (iota_nl == tgt_ref[b:b + 1, :]).astype(jnp.float32)
            ohf1 = (iota_nl == tgt_ref[b + 1:b + 2, :]).astype(jnp.float32)
            oh0 = ohf0.astype(_BF)
            oh1 = ohf1.astype(_BF)
            oh_scr[b] = jnp.concatenate([oh0, zcol], axis=1)
            oh_scr[b + 1] = jnp.concatenate([oh1, zcol], axis=1)
            msr = (b // 2) % 2
            pltpu.matmul_push_rhs(jnp.concatenate([oh0, oh1], axis=1),
                                  staging_register=msr, mxu_index=1)
            pltpu.matmul_acc_lhs(64 * msr, tr_bf, 1, load_staged_rhs=msr)
            apair = pltpu.matmul_pop(64 * msr, (_N, _N), _F32, 1)
            for k, ohf in ((0, ohf0), (1, ohf1)):
                a = apair[:, k * _L:(k + 1) * _L]
                tself[b + k:b + k + 1, :] = jnp.sum(ohf * a, axis=0,
                                                    keepdims=True)
                a_shift = jnp.concatenate([zcol_f, a[:, :-1]], axis=1)
                tm = jnp.sum(ohf * a_shift, axis=0, keepdims=True)
                tmove[b + k:b + k + 1, :] = jnp.where(iota_l == 0, _NEG, tm)
        for b in range(_BH):
            xslab = jnp.concatenate(
                [xb0_ref[b].astype(_BF), zeros8], axis=0)
            e = _emit_dot(xslab, oh_scr[b], b % 2, 32 * (b % 2))
            emits[0, :, b, :] = e[:_TC, :_L]
        x0 = xt_ref[0].astype(jnp.float32)
        z0 = jnp.max(x0, axis=1, keepdims=True)
        wfcc[...] = jnp.exp(x0 - z0)
        zfcc[...] = z0
        iota_bl = jax.lax.broadcasted_iota(jnp.int32, (_BH, _L), 1)
        afac[...] = jnp.where(iota_bl == 0, emits[0, 0], _NEG)

    sel = jax.lax.rem(t_idx, 2)
    sel_next = jax.lax.rem(t_idx + 1, 2)
    ts_mat = tself[...]
    tm_mat = tmove[...]
    neg_col = jnp.full((_BH, 1), _NEG, jnp.float32)

    w = wfcc[...].astype(_BF)
    z = zfcc[...]
    a_fac = afac[...]

    n_groups = _TC // 8
    splits = [(_BH * g) // n_groups for g in range(n_groups + 1)]

    for g in range(n_groups):
        xg = xt_ref[g * 8:(g + 1) * 8]
        for j in range(8):
            pltpu.matmul_acc_lhs(0, w, 0, load_staged_rhs=None)
            q = pltpu.matmul_pop(0, (_BH, _N), _F32, 0)
            scale = jnp.exp(xg[j].astype(jnp.float32))
            if j % 4 == 3:
                scale = scale * (1.0 / rs)
                z = z + jnp.log(rs)
            new_w = (scale * q).astype(_BF)
            stay = a_fac + ts_mat
            move = jnp.concatenate([neg_col, a_fac[:, :-1]], axis=1) + tm_mat
            mx = jnp.maximum(stay, move)
            new_fac = (mx + jnp.log1p(jnp.exp(-jnp.abs(stay - move)))
                       + emits[sel, g * 8 + j])
            if g == 0 and j == 0:
                new_w = jnp.where(t_idx == 0, w, new_w)
                new_fac = jnp.where(t_idx == 0, a_fac, new_fac)
            w, a_fac = new_w, new_fac
            if j % 4 == 2:
                rs = jnp.sum(w.astype(jnp.float32), axis=1, keepdims=True)
        for b in range(splits[g], splits[g + 1]):
            xslab = jnp.concatenate([xb_ref[b].astype(_BF), zeros8], axis=0)
            e = _emit_dot(xslab, oh_scr[b], b % 2, 32 * (b % 2))
            emits[sel_next, :, b, :] = e[:_TC, :_L]

    wfcc[...] = w.astype(jnp.float32)
    zfcc[...] = z
    afac[...] = a_fac

    @pl.when(t_idx == _NT - 1)
    def _finish():
        gmax = gmax_s[0, 0]
        fcc = (z + (_T - 1) * gmax
               + jnp.log(jnp.sum(w.astype(jnp.float32), axis=1,
                                 keepdims=True)))
        iota_bl = jax.lax.broadcasted_iota(jnp.int32, (_BH, _L), 1)
        mask = iota_bl == (ts_ref[...] - 1)
        fac = jnp.sum(jnp.where(mask, a_fac, 0.0), axis=1, keepdims=True)
        out_ref[...] = fcc - fac


def _asg_loss(x, target, target_size, trans):
    xt = jnp.moveaxis(x, 1, 0).astype(jnp.bfloat16)
    ts2 = target_size.reshape(_B, 1)
    out = pl.pallas_call(
        _asg_kernel,
        grid=(_NT,),
        in_specs=[
            pl.BlockSpec((_TC, _BH, _N), lambda t: (t, 0, 0)),
            pl.BlockSpec((_BH, _TC, _N),
                         lambda t: (0, jnp.minimum(t + 1, _NT - 1), 0)),
            pl.BlockSpec((_BH, _TC, _N), lambda t: (0, 0, 0)),
            pl.BlockSpec((_BH, _L), lambda t: (0, 0)),
            pl.BlockSpec((_BH, 1), lambda t: (0, 0)),
            pl.BlockSpec((_N, _N), lambda t: (0, 0)),
        ],
        out_specs=pl.BlockSpec((_BH, 1), lambda t: (0, 0)),
        out_shape=jax.ShapeDtypeStruct((_B, 1), jnp.float32),
        scratch_shapes=[
            pltpu.VMEM((_BH, _N), jnp.float32),
            pltpu.VMEM((_BH, 1), jnp.float32),
            pltpu.VMEM((_BH, _L), jnp.float32),
            pltpu.VMEM((_BH, _L), jnp.float32),
            pltpu.VMEM((_BH, _L), jnp.float32),
            pltpu.VMEM((2, _TC, _BH, _L), jnp.float32),
            pltpu.VMEM((_BH, _N, _N), _BF),
            pltpu.SMEM((1, 1), jnp.float32),
        ],
        compiler_params=pltpu.CompilerParams(
            dimension_semantics=("arbitrary",),
            vmem_limit_bytes=56 * 1024 * 1024,
        ),
    )(xt, x, x, target, ts2, trans)
    return out[:, 0]


def kernel(input, target, target_size, trans):
    return jax.jit(_asg_loss)(input, target, target_size, trans)
